# baseline (speedup 1.0000x reference)
import functools

import jax
import jax.numpy as jnp
from jax import lax
from jax.experimental import pallas as pl
from jax.experimental.pallas import tpu as pltpu
from jax.experimental.pallas import tpu_sc as plsc

NC = 2
NS = 16
NW = NC * NS


def _pre_body(x_ref, ws_ref, wr_ref, xs_ref, xr_ref):
    x = x_ref[...]
    xs_ref[...] = jnp.dot(x, ws_ref[...], preferred_element_type=jnp.float32)
    xr_ref[...] = jnp.dot(x, wr_ref[...], preferred_element_type=jnp.float32)


def _pre_tables(x, ws, wr, bn):
    n, d = x.shape
    grid = n // bn
    return pl.pallas_call(
        _pre_body,
        grid=(grid,),
        in_specs=[
            pl.BlockSpec((bn, d), lambda i: (i, 0)),
            pl.BlockSpec((d, d), lambda i: (0, 0)),
            pl.BlockSpec((d, d), lambda i: (0, 0)),
        ],
        out_specs=[
            pl.BlockSpec((bn, d), lambda i: (i, 0)),
            pl.BlockSpec((bn, d), lambda i: (i, 0)),
        ],
        out_shape=[
            jax.ShapeDtypeStruct((n, d), jnp.float32),
            jax.ShapeDtypeStruct((n, d), jnp.float32),
        ],
    )(x, ws, wr)


def _gather_sc(xs, xr, snd2d, rcv2d, tok, nchunks, chunk, gpc):
    n, d = xs.shape
    epw = nchunks * chunk
    e_total = NW * epw
    grows = gpc * chunk
    ngroups = nchunks // gpc
    npairs = ngroups // 2
    ntail = nchunks - ngroups * gpc
    mesh = plsc.VectorSubcoreMesh(core_axis_name="c", subcore_axis_name="s")

    @functools.partial(
        pl.kernel,
        mesh=mesh,
        out_type=jax.ShapeDtypeStruct((e_total, d), jnp.float32),
        scratch_types=[
            pltpu.VMEM((nchunks, chunk), jnp.int32),
            pltpu.VMEM((nchunks, chunk), jnp.int32),
            pltpu.VMEM((grows, d), jnp.float32),
            pltpu.VMEM((grows, d), jnp.float32),
            pltpu.VMEM((grows, d), jnp.float32),
            pltpu.VMEM((grows, d), jnp.float32),
            pltpu.SemaphoreType.DMA,
            pltpu.SemaphoreType.DMA,
        ],
    )
    def gather_k(xs_hbm, xr_hbm, snd_hbm, rcv_hbm, tok_hbm, g_hbm,
                 idx_s, idx_r, sb0, rb0, sb1, rb1, sem0, sem1):
        del tok_hbm
        wid = lax.axis_index("s") * NC + lax.axis_index("c")
        base = wid * epw
        pltpu.sync_copy(snd_hbm.at[wid], idx_s)
        pltpu.sync_copy(rcv_hbm.at[wid], idx_r)

        def issue(g, sb, rb, sem):
            for t in range(gpc):
                sl = pl.ds(t * chunk, chunk)
                pltpu.async_copy(xs_hbm.at[idx_s.at[gpc * g + t]],
                                 sb.at[sl], sem)
                pltpu.async_copy(xr_hbm.at[idx_r.at[gpc * g + t]],
                                 rb.at[sl], sem)

        def drain(sb, rb, sem):
            for t in range(gpc):
                sl = pl.ds(t * chunk, chunk)
                pltpu.make_async_copy(
                    xs_hbm.at[idx_s.at[0]], sb.at[sl], sem).wait()
                pltpu.make_async_copy(
                    xr_hbm.at[idx_r.at[0]], rb.at[sl], sem).wait()

        def addrows(sb, rb, nrows):
            def rbody(r, _):
                for cidx in range(d // 16):
                    sl = pl.ds(cidx * 16, 16)
                    sb[r, sl] = sb[r, sl] + rb[r, sl]
                return 0

            lax.fori_loop(0, nrows, rbody, 0)

        issue(0, sb0, rb0, sem0)

        def body(i, _):
            g0 = 2 * i
            g1 = 2 * i + 1
            drain(sb0, rb0, sem0)
            issue(g1, sb1, rb1, sem1)
            addrows(sb0, rb0, grows)
            pltpu.sync_copy(sb0, g_hbm.at[pl.ds(base + g0 * grows, grows)])
            drain(sb1, rb1, sem1)
            issue(jnp.minimum(g0 + 2, ngroups - 1), sb0, rb0, sem0)
            addrows(sb1, rb1, grows)
            pltpu.sync_copy(sb1, g_hbm.at[pl.ds(base + g1 * grows, grows)])
            return 0

        lax.fori_loop(0, npairs, body, 0)
        drain(sb0, rb0, sem0)
        if ngroups % 2 == 1:
            addrows(sb0, rb0, grows)
            pltpu.sync_copy(
                sb0, g_hbm.at[pl.ds(base + (ngroups - 1) * grows, grows)])
        for t in range(ntail):
            j = nchunks - ntail + t
            pltpu.async_copy(xs_hbm.at[idx_s.at[j]],
                             sb0.at[pl.ds(0, chunk)], sem0)
            pltpu.async_copy(xr_hbm.at[idx_r.at[j]],
                             rb0.at[pl.ds(0, chunk)], sem0)
            pltpu.make_async_copy(
                xs_hbm.at[idx_s.at[0]], sb0.at[pl.ds(0, chunk)], sem0).wait()
            pltpu.make_async_copy(
                xr_hbm.at[idx_r.at[0]], rb0.at[pl.ds(0, chunk)], sem0).wait()
            addrows(sb0, rb0, chunk)
            pltpu.sync_copy(sb0.at[pl.ds(0, chunk)],
                            g_hbm.at[pl.ds(base + j * chunk, chunk)])

    return gather_k(xs, xr, snd2d, rcv2d, tok)


def _edge_body(g_ref, e_ref, we_ref, b1_ref, w2_ref, b2_ref,
               gg_ref, bb_ref, ne_ref, res_ref):
    e = e_ref[...]
    h = g_ref[...] + b1_ref[...]
    h = h + jnp.dot(e, we_ref[...], preferred_element_type=jnp.float32)
    h = jnp.maximum(h, 0.0)
    h2 = jnp.dot(h, w2_ref[...], preferred_element_type=jnp.float32) + b2_ref[...]
    mu = jnp.mean(h2, axis=-1, keepdims=True)
    dlt = h2 - mu
    var = jnp.mean(dlt * dlt, axis=-1, keepdims=True)
    ne = dlt * lax.rsqrt(var + 1e-5) * gg_ref[...] + bb_ref[...]
    ne_ref[...] = ne
    res_ref[...] = ne + e


def _edge_mlp(g, e, we, b1, w2, b2, gg, bb, be, e_total, blk0, res_in):
    eh, d = g.shape
    grid = eh // be
    vec = lambda i: (0, 0)
    blk = lambda i: (i, 0)
    in_specs = [
        pl.BlockSpec((be, d), blk),
        pl.BlockSpec((be, d), lambda i: (i + blk0, 0)),
        pl.BlockSpec((d, d), vec),
        pl.BlockSpec((1, d), vec),
        pl.BlockSpec((d, d), vec),
        pl.BlockSpec((1, d), vec),
        pl.BlockSpec((1, d), vec),
        pl.BlockSpec((1, d), vec),
    ]
    args = [g, e, we, b1, w2, b2, gg, bb]
    kwargs = {}
    body = _edge_body
    if res_in is not None:
        in_specs.append(pl.BlockSpec(memory_space=pl.ANY))
        args.append(res_in)
        kwargs["input_output_aliases"] = {8: 1}
        body = lambda *refs: _edge_body(*refs[:8], *refs[9:])
    return pl.pallas_call(
        body,
        grid=(grid,),
        in_specs=in_specs,
        out_specs=[
            pl.BlockSpec((be, d), blk),
            pl.BlockSpec((be, d), lambda i: (i + blk0, 0)),
        ],
        out_shape=[
            jax.ShapeDtypeStruct((eh, d), jnp.float32),
            jax.ShapeDtypeStruct((e_total, d), jnp.float32),
        ],
        **kwargs,
    )(*args)


def _scatter_sc(ne, rcv2d, zeros_hbm, tok, n_pad, nchunks, chunk):
    _, d = ne.shape
    epw = nchunks * chunk
    rows_per_tile = n_pad // NS
    dc = 80
    mesh = plsc.VectorSubcoreMesh(core_axis_name="c", subcore_axis_name="s")

    @functools.partial(
        pl.kernel,
        mesh=mesh,
        out_type=jax.ShapeDtypeStruct((NC * n_pad, d), jnp.float32),
        scratch_types=[
            pltpu.VMEM((nchunks, chunk), jnp.int32),
            pltpu.VMEM((2, chunk, d), jnp.float32),
            pltpu.VMEM_SHARED((n_pad, d), jnp.float32),
            pltpu.SemaphoreType.DMA,
            pltpu.SemaphoreType.DMA,
        ],
    )
    def scatter_k(ne_hbm, rcv_hbm, z_hbm, tok_hbm, out_hbm, idx_v, rows, agg_sh,
                  sem0, sem1):
        del tok_hbm
        c = lax.axis_index("c")
        s = lax.axis_index("s")
        wid = s * NC + c
        base = wid * epw
        r0 = s * rows_per_tile
        nz = rows_per_tile // dc
        zrows = rows.at[0].at[pl.ds(0, dc)]
        pltpu.sync_copy(z_hbm, zrows)

        def zbody(k, _):
            pltpu.sync_copy(zrows, agg_sh.at[pl.ds(r0 + k * dc, dc)])
            return 0

        lax.fori_loop(0, nz, zbody, 0)
        plsc.subcore_barrier()
        pltpu.sync_copy(rcv_hbm.at[wid], idx_v)
        nch = lambda j: ne_hbm.at[pl.ds(base + j * chunk, chunk)]
        pltpu.async_copy(nch(0), rows.at[0], sem0)

        def body(i, _):
            j = 2 * i
            pltpu.async_copy(nch(j + 1), rows.at[1], sem1)
            pltpu.make_async_copy(nch(0), rows.at[0], sem0).wait()
            pltpu.sync_copy(rows.at[0], agg_sh.at[idx_v.at[j]], add=True)
            pltpu.async_copy(nch(j + 2), rows.at[0], sem0)
            pltpu.make_async_copy(nch(0), rows.at[1], sem1).wait()
            pltpu.sync_copy(rows.at[1], agg_sh.at[idx_v.at[j + 1]], add=True)
            return 0

        if nchunks % 2 == 1:
            lax.fori_loop(0, (nchunks - 1) // 2, body, 0)
            pltpu.make_async_copy(nch(0), rows.at[0], sem0).wait()
            pltpu.sync_copy(rows.at[0], agg_sh.at[idx_v.at[nchunks - 1]],
                            add=True)
        else:
            lax.fori_loop(0, (nchunks - 2) // 2, body, 0)
            pltpu.async_copy(nch(nchunks - 1), rows.at[1], sem1)
            pltpu.make_async_copy(nch(0), rows.at[0], sem0).wait()
            pltpu.sync_copy(rows.at[0], agg_sh.at[idx_v.at[nchunks - 2]],
                            add=True)
            pltpu.make_async_copy(nch(0), rows.at[1], sem1).wait()
            pltpu.sync_copy(rows.at[1], agg_sh.at[idx_v.at[nchunks - 1]],
                            add=True)
        plsc.subcore_barrier()

        def dbody(k, _):
            pltpu.sync_copy(agg_sh.at[pl.ds(r0 + k * dc, dc)], zrows)
            pltpu.sync_copy(
                zrows, out_hbm.at[pl.ds(c * n_pad + r0 + k * dc, dc)])
            return 0

        lax.fori_loop(0, nz, dbody, 0)

    return scatter_k(ne, rcv2d, zeros_hbm, tok)


def _node_body(x_ref, a0_ref, a1_ref, a2_ref, a3_ref, wx_ref, wa_ref, b1_ref,
               w2_ref, b2_ref, g_ref, b_ref, out_ref):
    x = x_ref[...]
    agg = (a0_ref[...] + a1_ref[...]) + (a2_ref[...] + a3_ref[...])
    h = jnp.dot(x, wx_ref[...], preferred_element_type=jnp.float32)
    h = h + jnp.dot(agg, wa_ref[...], preferred_element_type=jnp.float32)
    h = jnp.maximum(h + b1_ref[...], 0.0)
    h2 = jnp.dot(h, w2_ref[...], preferred_element_type=jnp.float32) + b2_ref[...]
    mu = jnp.mean(h2, axis=-1, keepdims=True)
    dlt = h2 - mu
    var = jnp.mean(dlt * dlt, axis=-1, keepdims=True)
    out_ref[...] = dlt * lax.rsqrt(var + 1e-5) * g_ref[...] + b_ref[...] + x


def _node_mlp(x, aggs, wx, wa, b1, w2, b2, g, b, bn):
    n, d = x.shape
    grid = n // bn
    vec = lambda i: (0, 0)
    blk = pl.BlockSpec((bn, d), lambda i: (i, 0))
    return pl.pallas_call(
        _node_body,
        grid=(grid,),
        in_specs=[blk, blk, blk, blk, blk,
                  pl.BlockSpec((d, d), vec),
                  pl.BlockSpec((d, d), vec),
                  pl.BlockSpec((1, d), vec),
                  pl.BlockSpec((d, d), vec),
                  pl.BlockSpec((1, d), vec),
                  pl.BlockSpec((1, d), vec),
                  pl.BlockSpec((1, d), vec)],
        out_specs=pl.BlockSpec((bn, d), lambda i: (i, 0)),
        out_shape=jax.ShapeDtypeStruct((n, d), jnp.float32),
    )(x, *aggs, wx, wa, b1, w2, b2, g, b)


def kernel(node_features, edge_features, senders, receivers,
           We1, be1, We2, be2, eg, eb, Wn1, bn1, Wn2, bn2, ng, nb):
    n, d = node_features.shape
    e_total = edge_features.shape[0]

    chunk = 80
    be = 8000
    e0, e1 = 192000, 128000

    we1s, we1r, we1e = We1[:d], We1[d:2 * d], We1[2 * d:]
    wn1x, wn1a = Wn1[:d], Wn1[d:]
    r2 = lambda v: v.reshape(1, d)

    n_pad = 10240
    zeros_hbm = jnp.zeros((chunk, d), jnp.float32)

    gck0 = 120
    nck0 = e0 // NW // gck0
    nck1 = e1 // NW // chunk
    snd0 = lax.slice(senders, (0,), (e0,)).reshape(NW, nck0, gck0)
    rcv0 = lax.slice(receivers, (0,), (e0,)).reshape(NW, nck0, gck0)
    snd1 = lax.slice(senders, (e0,), (e_total,)).reshape(NW, nck1, chunk)
    rcv1 = lax.slice(receivers, (e0,), (e_total,)).reshape(NW, nck1, chunk)

    xs, xr = _pre_tables(node_features, we1s, we1r, bn=2000)
    g0 = _gather_sc(xs, xr, snd0, rcv0, xs[:8], nck0, gck0, 1)
    g1 = _gather_sc(xs, xr, snd1, rcv1, g0[:8], nck1, chunk, 2)
    ne0, res0 = _edge_mlp(g0, edge_features, we1e, r2(be1), We2, r2(be2),
                          r2(eg), r2(eb), be, e_total, 0, None)
    sck0 = 120
    rcv0s = lax.slice(receivers, (0,), (e0,)).reshape(NW, e0 // NW // sck0, sck0)
    agg0 = _scatter_sc(ne0, rcv0s, zeros_hbm, g1[:8], n_pad,
                       e0 // NW // sck0, sck0)
    ne1, new_e = _edge_mlp(g1, edge_features, we1e, r2(be1), We2, r2(be2),
                           r2(eg), r2(eb), be, e_total, e0 // be, res0)
    agg1 = _scatter_sc(ne1, rcv1, zeros_hbm, agg0[:8], n_pad, nck1, chunk)
    a = [p[o:o + n] for p in (agg0, agg1) for o in (0, n_pad)]
    new_x = _node_mlp(node_features, a, wn1x, wn1a, r2(bn1),
                      Wn2, r2(bn2), r2(ng), r2(nb), bn=5000)
    return new_x, new_e


# speedup vs baseline: 1.0019x; 1.0019x over previous
import functools

import jax
import jax.numpy as jnp
from jax import lax
from jax.experimental import pallas as pl
from jax.experimental.pallas import tpu as pltpu
from jax.experimental.pallas import tpu_sc as plsc

NC = 2
NS = 16
NW = NC * NS


def _pre_body(x_ref, ws_ref, wr_ref, xs_ref, xr_ref):
    x = x_ref[...]
    xs_ref[...] = jnp.dot(x, ws_ref[...], preferred_element_type=jnp.float32)
    xr_ref[...] = jnp.dot(x, wr_ref[...], preferred_element_type=jnp.float32)


def _pre_tables(x, ws, wr, bn):
    n, d = x.shape
    grid = n // bn
    return pl.pallas_call(
        _pre_body,
        grid=(grid,),
        in_specs=[
            pl.BlockSpec((bn, d), lambda i: (i, 0)),
            pl.BlockSpec((d, d), lambda i: (0, 0)),
            pl.BlockSpec((d, d), lambda i: (0, 0)),
        ],
        out_specs=[
            pl.BlockSpec((bn, d), lambda i: (i, 0)),
            pl.BlockSpec((bn, d), lambda i: (i, 0)),
        ],
        out_shape=[
            jax.ShapeDtypeStruct((n, d), jnp.float32),
            jax.ShapeDtypeStruct((n, d), jnp.float32),
        ],
    )(x, ws, wr)


def _gather_sc(xs, xr, snd2d, rcv2d, tok, nchunks, chunk):
    n, d = xs.shape
    epw = nchunks * chunk
    e_total = NW * epw
    grows = 2 * chunk
    ngroups = nchunks // 2
    npairs = ngroups // 2
    mesh = plsc.VectorSubcoreMesh(core_axis_name="c", subcore_axis_name="s")

    @functools.partial(
        pl.kernel,
        mesh=mesh,
        out_type=jax.ShapeDtypeStruct((e_total, d), jnp.float32),
        scratch_types=[
            pltpu.VMEM((nchunks, chunk), jnp.int32),
            pltpu.VMEM((nchunks, chunk), jnp.int32),
            pltpu.VMEM((grows, d), jnp.float32),
            pltpu.VMEM((grows, d), jnp.float32),
            pltpu.VMEM((grows, d), jnp.float32),
            pltpu.VMEM((grows, d), jnp.float32),
            pltpu.SemaphoreType.DMA,
            pltpu.SemaphoreType.DMA,
        ],
    )
    def gather_k(xs_hbm, xr_hbm, snd_hbm, rcv_hbm, tok_hbm, g_hbm,
                 idx_s, idx_r, sb0, rb0, sb1, rb1, sem0, sem1):
        del tok_hbm
        wid = lax.axis_index("s") * NC + lax.axis_index("c")
        base = wid * epw
        pltpu.sync_copy(snd_hbm.at[wid], idx_s)
        pltpu.sync_copy(rcv_hbm.at[wid], idx_r)

        def issue(g, sb, rb, sem):
            pltpu.async_copy(xs_hbm.at[idx_s.at[2 * g]],
                             sb.at[pl.ds(0, chunk)], sem)
            pltpu.async_copy(xs_hbm.at[idx_s.at[2 * g + 1]],
                             sb.at[pl.ds(chunk, chunk)], sem)
            pltpu.async_copy(xr_hbm.at[idx_r.at[2 * g]],
                             rb.at[pl.ds(0, chunk)], sem)
            pltpu.async_copy(xr_hbm.at[idx_r.at[2 * g + 1]],
                             rb.at[pl.ds(chunk, chunk)], sem)

        def drain(sb, rb, sem):
            pltpu.make_async_copy(
                xs_hbm.at[idx_s.at[0]], sb.at[pl.ds(0, chunk)], sem).wait()
            pltpu.make_async_copy(
                xs_hbm.at[idx_s.at[0]], sb.at[pl.ds(chunk, chunk)], sem).wait()
            pltpu.make_async_copy(
                xr_hbm.at[idx_r.at[0]], rb.at[pl.ds(0, chunk)], sem).wait()
            pltpu.make_async_copy(
                xr_hbm.at[idx_r.at[0]], rb.at[pl.ds(chunk, chunk)], sem).wait()

        def addrows(sb, rb, nrows):
            def rbody(r, _):
                for cidx in range(d // 16):
                    sl = pl.ds(cidx * 16, 16)
                    sb[r, sl] = sb[r, sl] + rb[r, sl]
                return 0

            lax.fori_loop(0, nrows, rbody, 0)

        issue(0, sb0, rb0, sem0)

        def body(i, _):
            g0 = 2 * i
            g1 = 2 * i + 1
            drain(sb0, rb0, sem0)
            issue(g1, sb1, rb1, sem1)
            addrows(sb0, rb0, grows)
            pltpu.sync_copy(sb0, g_hbm.at[pl.ds(base + g0 * grows, grows)])
            drain(sb1, rb1, sem1)
            issue(jnp.minimum(g0 + 2, ngroups - 1), sb0, rb0, sem0)
            addrows(sb1, rb1, grows)
            pltpu.sync_copy(sb1, g_hbm.at[pl.ds(base + g1 * grows, grows)])
            return 0

        lax.fori_loop(0, npairs, body, 0)
        drain(sb0, rb0, sem0)
        if ngroups % 2 == 1:
            addrows(sb0, rb0, grows)
            pltpu.sync_copy(
                sb0, g_hbm.at[pl.ds(base + (ngroups - 1) * grows, grows)])
        if nchunks % 2 == 1:
            j = nchunks - 1
            pltpu.async_copy(xs_hbm.at[idx_s.at[j]],
                             sb0.at[pl.ds(0, chunk)], sem0)
            pltpu.async_copy(xr_hbm.at[idx_r.at[j]],
                             rb0.at[pl.ds(0, chunk)], sem0)
            pltpu.make_async_copy(
                xs_hbm.at[idx_s.at[0]], sb0.at[pl.ds(0, chunk)], sem0).wait()
            pltpu.make_async_copy(
                xr_hbm.at[idx_r.at[0]], rb0.at[pl.ds(0, chunk)], sem0).wait()
            addrows(sb0, rb0, chunk)
            pltpu.sync_copy(sb0.at[pl.ds(0, chunk)],
                            g_hbm.at[pl.ds(base + j * chunk, chunk)])

    return gather_k(xs, xr, snd2d, rcv2d, tok)


def _edge_body(g_ref, e_ref, we_ref, b1_ref, w2_ref, b2_ref,
               gg_ref, bb_ref, ne_ref, res_ref):
    e = e_ref[...]
    h = g_ref[...] + b1_ref[...]
    h = h + jnp.dot(e, we_ref[...], preferred_element_type=jnp.float32)
    h = jnp.maximum(h, 0.0)
    h2 = jnp.dot(h, w2_ref[...], preferred_element_type=jnp.float32) + b2_ref[...]
    mu = jnp.mean(h2, axis=-1, keepdims=True)
    dlt = h2 - mu
    var = jnp.mean(dlt * dlt, axis=-1, keepdims=True)
    ne = dlt * lax.rsqrt(var + 1e-5) * gg_ref[...] + bb_ref[...]
    ne_ref[...] = ne
    res_ref[...] = ne + e


def _edge_mlp(g, e, we, b1, w2, b2, gg, bb, be, e_total, blk0, res_in):
    eh, d = g.shape
    grid = eh // be
    vec = lambda i: (0, 0)
    blk = lambda i: (i, 0)
    in_specs = [
        pl.BlockSpec((be, d), blk),
        pl.BlockSpec((be, d), lambda i: (i + blk0, 0)),
        pl.BlockSpec((d, d), vec),
        pl.BlockSpec((1, d), vec),
        pl.BlockSpec((d, d), vec),
        pl.BlockSpec((1, d), vec),
        pl.BlockSpec((1, d), vec),
        pl.BlockSpec((1, d), vec),
    ]
    args = [g, e, we, b1, w2, b2, gg, bb]
    kwargs = {}
    body = _edge_body
    if res_in is not None:
        in_specs.append(pl.BlockSpec(memory_space=pl.ANY))
        args.append(res_in)
        kwargs["input_output_aliases"] = {8: 1}
        body = lambda *refs: _edge_body(*refs[:8], *refs[9:])
    return pl.pallas_call(
        body,
        grid=(grid,),
        in_specs=in_specs,
        out_specs=[
            pl.BlockSpec((be, d), blk),
            pl.BlockSpec((be, d), lambda i: (i + blk0, 0)),
        ],
        out_shape=[
            jax.ShapeDtypeStruct((eh, d), jnp.float32),
            jax.ShapeDtypeStruct((e_total, d), jnp.float32),
        ],
        **kwargs,
    )(*args)


def _scatter_sc(ne, rcv2d, zeros_hbm, tok, n_pad, nchunks, chunk):
    _, d = ne.shape
    epw = nchunks * chunk
    rows_per_tile = n_pad // NS
    dc = 80
    mesh = plsc.VectorSubcoreMesh(core_axis_name="c", subcore_axis_name="s")

    @functools.partial(
        pl.kernel,
        mesh=mesh,
        out_type=jax.ShapeDtypeStruct((NC * n_pad, d), jnp.float32),
        scratch_types=[
            pltpu.VMEM((nchunks, chunk), jnp.int32),
            pltpu.VMEM((2, chunk, d), jnp.float32),
            pltpu.VMEM_SHARED((n_pad, d), jnp.float32),
            pltpu.SemaphoreType.DMA,
            pltpu.SemaphoreType.DMA,
        ],
    )
    def scatter_k(ne_hbm, rcv_hbm, z_hbm, tok_hbm, out_hbm, idx_v, rows, agg_sh,
                  sem0, sem1):
        del tok_hbm
        c = lax.axis_index("c")
        s = lax.axis_index("s")
        wid = s * NC + c
        base = wid * epw
        r0 = s * rows_per_tile
        nz = rows_per_tile // dc
        zrows = rows.at[0].at[pl.ds(0, dc)]
        pltpu.sync_copy(z_hbm, zrows)

        def zbody(k, _):
            pltpu.sync_copy(zrows, agg_sh.at[pl.ds(r0 + k * dc, dc)])
            return 0

        lax.fori_loop(0, nz, zbody, 0)
        plsc.subcore_barrier()
        pltpu.sync_copy(rcv_hbm.at[wid], idx_v)
        nch = lambda j: ne_hbm.at[pl.ds(base + j * chunk, chunk)]
        pltpu.async_copy(nch(0), rows.at[0], sem0)

        def body(i, _):
            j = 2 * i
            pltpu.async_copy(nch(j + 1), rows.at[1], sem1)
            pltpu.make_async_copy(nch(0), rows.at[0], sem0).wait()
            pltpu.sync_copy(rows.at[0], agg_sh.at[idx_v.at[j]], add=True)
            pltpu.async_copy(nch(j + 2), rows.at[0], sem0)
            pltpu.make_async_copy(nch(0), rows.at[1], sem1).wait()
            pltpu.sync_copy(rows.at[1], agg_sh.at[idx_v.at[j + 1]], add=True)
            return 0

        if nchunks % 2 == 1:
            lax.fori_loop(0, (nchunks - 1) // 2, body, 0)
            pltpu.make_async_copy(nch(0), rows.at[0], sem0).wait()
            pltpu.sync_copy(rows.at[0], agg_sh.at[idx_v.at[nchunks - 1]],
                            add=True)
        else:
            lax.fori_loop(0, (nchunks - 2) // 2, body, 0)
            pltpu.async_copy(nch(nchunks - 1), rows.at[1], sem1)
            pltpu.make_async_copy(nch(0), rows.at[0], sem0).wait()
            pltpu.sync_copy(rows.at[0], agg_sh.at[idx_v.at[nchunks - 2]],
                            add=True)
            pltpu.make_async_copy(nch(0), rows.at[1], sem1).wait()
            pltpu.sync_copy(rows.at[1], agg_sh.at[idx_v.at[nchunks - 1]],
                            add=True)
        plsc.subcore_barrier()

        def dbody(k, _):
            pltpu.sync_copy(agg_sh.at[pl.ds(r0 + k * dc, dc)], zrows)
            pltpu.sync_copy(
                zrows, out_hbm.at[pl.ds(c * n_pad + r0 + k * dc, dc)])
            return 0

        lax.fori_loop(0, nz, dbody, 0)

    return scatter_k(ne, rcv2d, zeros_hbm, tok)


def _node_body(x_ref, a0_ref, a1_ref, a2_ref, a3_ref, wx_ref, wa_ref, b1_ref,
               w2_ref, b2_ref, g_ref, b_ref, out_ref):
    x = x_ref[...]
    agg = (a0_ref[...] + a1_ref[...]) + (a2_ref[...] + a3_ref[...])
    h = jnp.dot(x, wx_ref[...], preferred_element_type=jnp.float32)
    h = h + jnp.dot(agg, wa_ref[...], preferred_element_type=jnp.float32)
    h = jnp.maximum(h + b1_ref[...], 0.0)
    h2 = jnp.dot(h, w2_ref[...], preferred_element_type=jnp.float32) + b2_ref[...]
    mu = jnp.mean(h2, axis=-1, keepdims=True)
    dlt = h2 - mu
    var = jnp.mean(dlt * dlt, axis=-1, keepdims=True)
    out_ref[...] = dlt * lax.rsqrt(var + 1e-5) * g_ref[...] + b_ref[...] + x


def _node_mlp(x, aggs, wx, wa, b1, w2, b2, g, b, bn):
    n, d = x.shape
    grid = n // bn
    vec = lambda i: (0, 0)
    blk = pl.BlockSpec((bn, d), lambda i: (i, 0))
    return pl.pallas_call(
        _node_body,
        grid=(grid,),
        in_specs=[blk, blk, blk, blk, blk,
                  pl.BlockSpec((d, d), vec),
                  pl.BlockSpec((d, d), vec),
                  pl.BlockSpec((1, d), vec),
                  pl.BlockSpec((d, d), vec),
                  pl.BlockSpec((1, d), vec),
                  pl.BlockSpec((1, d), vec),
                  pl.BlockSpec((1, d), vec)],
        out_specs=pl.BlockSpec((bn, d), lambda i: (i, 0)),
        out_shape=jax.ShapeDtypeStruct((n, d), jnp.float32),
    )(x, *aggs, wx, wa, b1, w2, b2, g, b)


def kernel(node_features, edge_features, senders, receivers,
           We1, be1, We2, be2, eg, eb, Wn1, bn1, Wn2, bn2, ng, nb):
    n, d = node_features.shape
    e_total = edge_features.shape[0]

    chunk = 80
    be = 8000
    e0, e1 = 192000, 128000

    we1s, we1r, we1e = We1[:d], We1[d:2 * d], We1[2 * d:]
    wn1x, wn1a = Wn1[:d], Wn1[d:]
    r2 = lambda v: v.reshape(1, d)

    n_pad = 10240
    zeros_hbm = jnp.zeros((chunk, d), jnp.float32)

    nck0 = e0 // NW // chunk
    nck1 = e1 // NW // chunk
    snd0 = lax.slice(senders, (0,), (e0,)).reshape(NW, nck0, chunk)
    rcv0 = lax.slice(receivers, (0,), (e0,)).reshape(NW, nck0, chunk)
    snd1 = lax.slice(senders, (e0,), (e_total,)).reshape(NW, nck1, chunk)
    rcv1 = lax.slice(receivers, (e0,), (e_total,)).reshape(NW, nck1, chunk)
    xs, xr = _pre_tables(node_features, we1s, we1r, bn=2000)
    g0 = _gather_sc(xs, xr, snd0, rcv0, xs[:8], nck0, chunk)
    g1 = _gather_sc(xs, xr, snd1, rcv1, g0[:8], nck1, chunk)
    ne0, res0 = _edge_mlp(g0, edge_features, we1e, r2(be1), We2, r2(be2),
                          r2(eg), r2(eb), be, e_total, 0, None)
    sck0 = 120
    rcv0s = lax.slice(receivers, (0,), (e0,)).reshape(NW, e0 // NW // sck0, sck0)
    agg0 = _scatter_sc(ne0, rcv0s, zeros_hbm, g1[:8], n_pad,
                       e0 // NW // sck0, sck0)
    ne1, new_e = _edge_mlp(g1, edge_features, we1e, r2(be1), We2, r2(be2),
                           r2(eg), r2(eb), be, e_total, e0 // be, res0)
    agg1 = _scatter_sc(ne1, rcv1, zeros_hbm, agg0[:8], n_pad, nck1, chunk)
    a = [p[o:o + n] for p in (agg0, agg1) for o in (0, n_pad)]
    new_x = _node_mlp(node_features, a, wn1x, wn1a, r2(bn1),
                      Wn2, r2(bn2), r2(ng), r2(nb), bn=5000)
    return new_x, new_e
